# baseline (speedup 1.0000x reference)
import functools

import jax
import jax.numpy as jnp
from jax import lax
from jax.experimental import pallas as pl
from jax.experimental.pallas import tpu as pltpu
from jax.experimental.pallas import tpu_sc as plsc

BATCH = 16384
EMB = 64

_info = plsc.get_sparse_core_info()
_NC, _NS = _info.num_cores, _info.num_subcores
_NW = _NC * _NS
_BPW = BATCH // _NW


def _gather_body(wu_hbm, wi_hbm, u_hbm, i_hbm, eu_hbm, ei_hbm,
                 idx_u, idx_i, rows_u, rows_i, sem_u, sem_i):
    wid = lax.axis_index("s") * _NC + lax.axis_index("c")
    base = wid * _BPW
    pltpu.sync_copy(u_hbm.at[pl.ds(base, _BPW)], idx_u)
    pltpu.sync_copy(i_hbm.at[pl.ds(base, _BPW)], idx_i)
    cu = pltpu.async_copy(wu_hbm.at[idx_u], rows_u, sem_u)
    ci = pltpu.async_copy(wi_hbm.at[idx_i], rows_i, sem_i)
    cu.wait()
    pltpu.sync_copy(rows_u, eu_hbm.at[pl.ds(base, _BPW)])
    ci.wait()
    pltpu.sync_copy(rows_i, ei_hbm.at[pl.ds(base, _BPW)])


def _sc_gather(wu, wi, u, i):
    mesh = plsc.VectorSubcoreMesh(core_axis_name="c", subcore_axis_name="s")
    f = functools.partial(
        pl.kernel,
        mesh=mesh,
        out_type=(
            jax.ShapeDtypeStruct((BATCH, EMB), jnp.float32),
            jax.ShapeDtypeStruct((BATCH, EMB), jnp.float32),
        ),
        scratch_types=[
            pltpu.VMEM((_BPW,), jnp.int32),
            pltpu.VMEM((_BPW,), jnp.int32),
            pltpu.VMEM((_BPW, EMB), jnp.float32),
            pltpu.VMEM((_BPW, EMB), jnp.float32),
            pltpu.SemaphoreType.DMA,
            pltpu.SemaphoreType.DMA,
        ],
        compiler_params=pltpu.CompilerParams(use_tc_tiling_on_sc=False),
    )(_gather_body)
    return f(wu, wi, u, i)


def _mlp_body(eu_ref, ei_ref, w1a_ref, w1b_ref, b1_ref, w2_ref, b2_ref,
              w3_ref, b3_ref, out_ref):
    h = jnp.dot(eu_ref[...], w1a_ref[...], preferred_element_type=jnp.float32)
    h = h + jnp.dot(ei_ref[...], w1b_ref[...],
                    preferred_element_type=jnp.float32)
    h = jnp.maximum(h + b1_ref[...], 0.0)
    h = jnp.maximum(
        jnp.dot(h, w2_ref[...], preferred_element_type=jnp.float32)
        + b2_ref[...], 0.0)
    o = jnp.dot(h, w3_ref[...], preferred_element_type=jnp.float32) + b3_ref[...]
    out_ref[...] = jax.nn.sigmoid(o)


def _tc_mlp(eu, ei, w1, b1, w2, b2, w3, b3):
    blk = 2048
    grid = BATCH // blk
    w1a = w1[:EMB]
    w1b = w1[EMB:]
    full = lambda shape: pl.BlockSpec(shape, lambda g: (0, 0))
    out = pl.pallas_call(
        _mlp_body,
        grid=(grid,),
        in_specs=[
            pl.BlockSpec((blk, EMB), lambda g: (g, 0)),
            pl.BlockSpec((blk, EMB), lambda g: (g, 0)),
            full((EMB, 128)),
            full((EMB, 128)),
            full((1, 128)),
            full((128, 64)),
            full((1, 64)),
            full((64, 1)),
            full((1, 1)),
        ],
        out_specs=pl.BlockSpec((blk, 1), lambda g: (g, 0)),
        out_shape=jax.ShapeDtypeStruct((BATCH, 1), jnp.float32),
        compiler_params=pltpu.CompilerParams(
            dimension_semantics=("arbitrary",)),
    )(eu, ei, w1a, w1b, b1.reshape(1, 128), w2, b2.reshape(1, 64), w3,
      b3.reshape(1, 1))
    return jnp.squeeze(out, axis=-1)


def kernel(u, i, Wu, Wi, W1, b1, W2, b2, W3, b3):
    eu, ei = _sc_gather(Wu, Wi, u.astype(jnp.int32), i.astype(jnp.int32))
    return _tc_mlp(eu, ei, W1, b1, W2, b2, W3, b3)


# speedup vs baseline: 1.1793x; 1.1793x over previous
import functools

import jax
import jax.numpy as jnp
from jax import lax
from jax.experimental import pallas as pl
from jax.experimental.pallas import tpu as pltpu
from jax.experimental.pallas import tpu_sc as plsc

BATCH = 16384
EMB = 64
NROWS = 1000001
_TBLK = 1024
_TGRID = (NROWS + _TBLK - 1) // _TBLK
ZROWS = _TGRID * _TBLK

_info = plsc.get_sparse_core_info()
_NC, _NS = _info.num_cores, _info.num_subcores
_NW = _NC * _NS
_BPW = BATCH // _NW


def _transpose_body(wuT_ref, wiT_ref, eye_ref, z_ref):
    eye = eye_ref[...]
    dn = (((0,), (0,)), ((), ()))
    yu = lax.dot_general(wuT_ref[...], eye, dn,
                         preferred_element_type=jnp.float32)
    yi = lax.dot_general(wiT_ref[...], eye, dn,
                         preferred_element_type=jnp.float32)
    z_ref[...] = jnp.concatenate([yu, yi], axis=1)


def _tc_transpose(wuT, wiT):
    eye = jnp.eye(EMB, dtype=jnp.float32)
    return pl.pallas_call(
        _transpose_body,
        grid=(_TGRID,),
        in_specs=[
            pl.BlockSpec((EMB, _TBLK), lambda g: (0, g)),
            pl.BlockSpec((EMB, _TBLK), lambda g: (0, g)),
            pl.BlockSpec((EMB, EMB), lambda g: (0, 0)),
        ],
        out_specs=pl.BlockSpec((_TBLK, 2 * EMB), lambda g: (g, 0)),
        out_shape=jax.ShapeDtypeStruct((ZROWS, 2 * EMB), jnp.float32),
        compiler_params=pltpu.CompilerParams(
            dimension_semantics=("arbitrary",)),
    )(wuT, wiT, eye)


def _gather_body(z_hbm, u_hbm, i_hbm, eu_hbm, ei_hbm, idx_v, dst, sem):
    wid = lax.axis_index("s") * _NC + lax.axis_index("c")
    base = wid * _BPW

    def do_table(idx_hbm, out_ref):
        pltpu.sync_copy(idx_hbm.at[pl.ds(base, _BPW)], idx_v)
        copies = [
            pltpu.async_copy(z_hbm.at[idx_v.at[pl.ds(q * 128, 128)]],
                             dst.at[pl.ds(q * 128, 128)], sem)
            for q in range(_BPW // 128)
        ]
        for cp in copies:
            cp.wait()
        pltpu.sync_copy(dst, out_ref.at[pl.ds(base, _BPW)])

    do_table(u_hbm, eu_hbm)
    do_table(i_hbm, ei_hbm)


def _sc_gather(z, u, i):
    mesh = plsc.VectorSubcoreMesh(core_axis_name="c", subcore_axis_name="s")
    f = functools.partial(
        pl.kernel,
        mesh=mesh,
        out_type=(
            jax.ShapeDtypeStruct((BATCH, 2 * EMB), jnp.float32),
            jax.ShapeDtypeStruct((BATCH, 2 * EMB), jnp.float32),
        ),
        scratch_types=[
            pltpu.VMEM((_BPW,), jnp.int32),
            pltpu.VMEM((_BPW, 2 * EMB), jnp.float32),
            pltpu.SemaphoreType.DMA,
        ],
    )(_gather_body)
    return f(z, u, i)


def _mlp_body(eu_ref, ei_ref, w1a_ref, w1b_ref, b1_ref, w2_ref, b2_ref,
              w3_ref, b3_ref, out_ref):
    h = jnp.dot(eu_ref[...], w1a_ref[...], preferred_element_type=jnp.float32)
    h = h + jnp.dot(ei_ref[...], w1b_ref[...],
                    preferred_element_type=jnp.float32)
    h = jnp.maximum(h + b1_ref[...], 0.0)
    h = jnp.maximum(
        jnp.dot(h, w2_ref[...], preferred_element_type=jnp.float32)
        + b2_ref[...], 0.0)
    o = jnp.dot(h, w3_ref[...], preferred_element_type=jnp.float32) + b3_ref[...]
    out_ref[...] = jax.nn.sigmoid(o)


def _tc_mlp(eu, ei, w1, b1, w2, b2, w3, b3):
    blk = 2048
    grid = BATCH // blk
    zpad = jnp.zeros((EMB, 128), jnp.float32)
    w1a = jnp.concatenate([w1[:EMB], zpad], axis=0)
    w1b = jnp.concatenate([zpad, w1[EMB:]], axis=0)
    full = lambda shape: pl.BlockSpec(shape, lambda g: (0, 0))
    out = pl.pallas_call(
        _mlp_body,
        grid=(grid,),
        in_specs=[
            pl.BlockSpec((blk, 2 * EMB), lambda g: (g, 0)),
            pl.BlockSpec((blk, 2 * EMB), lambda g: (g, 0)),
            full((128, 128)),
            full((128, 128)),
            full((1, 128)),
            full((128, 64)),
            full((1, 64)),
            full((64, 1)),
            full((1, 1)),
        ],
        out_specs=pl.BlockSpec((blk, 1), lambda g: (g, 0)),
        out_shape=jax.ShapeDtypeStruct((BATCH, 1), jnp.float32),
        compiler_params=pltpu.CompilerParams(
            dimension_semantics=("arbitrary",)),
    )(eu, ei, w1a, w1b, b1.reshape(1, 128), w2, b2.reshape(1, 64), w3,
      b3.reshape(1, 1))
    return jnp.squeeze(out, axis=-1)


def kernel(u, i, Wu, Wi, W1, b1, W2, b2, W3, b3):
    z = _tc_transpose(Wu.T, Wi.T)
    eu, ei = _sc_gather(z, u.astype(jnp.int32), i.astype(jnp.int32))
    return _tc_mlp(eu, ei, W1, b1, W2, b2, W3, b3)


# speedup vs baseline: 2.6670x; 2.2616x over previous
import functools

import jax
import jax.numpy as jnp
from jax import lax
from jax.experimental import pallas as pl
from jax.experimental.pallas import tpu as pltpu
from jax.experimental.pallas import tpu_sc as plsc

BATCH = 16384
EMB = 64
NROWS = 1000001
_TBLK = 4096
_TGRID = (NROWS + _TBLK - 1) // _TBLK
ZROWS = _TGRID * _TBLK

_info = plsc.get_sparse_core_info()
_NC, _NS = _info.num_cores, _info.num_subcores
_NW = _NC * _NS
_BPW = BATCH // _NW


def _transpose_body(wuT_ref, wiT_ref, eye_ref, z_ref):
    x = jnp.concatenate(
        [wuT_ref[...], wiT_ref[...]], axis=0).astype(jnp.bfloat16)
    dn = (((0,), (0,)), ((), ()))
    z_ref[...] = lax.dot_general(x, eye_ref[...], dn,
                                 preferred_element_type=jnp.float32)


def _tc_transpose(wuT, wiT):
    eye = jnp.eye(2 * EMB, dtype=jnp.bfloat16)
    return pl.pallas_call(
        _transpose_body,
        grid=(_TGRID,),
        in_specs=[
            pl.BlockSpec((EMB, _TBLK), lambda g: (0, g)),
            pl.BlockSpec((EMB, _TBLK), lambda g: (0, g)),
            pl.BlockSpec((2 * EMB, 2 * EMB), lambda g: (0, 0)),
        ],
        out_specs=pl.BlockSpec((_TBLK, 2 * EMB), lambda g: (g, 0)),
        out_shape=jax.ShapeDtypeStruct((ZROWS, 2 * EMB), jnp.float32),
        compiler_params=pltpu.CompilerParams(
            dimension_semantics=("arbitrary",)),
    )(wuT, wiT, eye)


def _gather_body(z_hbm, u_hbm, i_hbm, eu_hbm, ei_hbm, idx_v, dst, sem):
    wid = lax.axis_index("s") * _NC + lax.axis_index("c")
    base = wid * _BPW

    def do_table(idx_hbm, out_ref):
        pltpu.sync_copy(idx_hbm.at[pl.ds(base, _BPW)], idx_v)
        copies = [
            pltpu.async_copy(z_hbm.at[idx_v.at[pl.ds(q * 128, 128)]],
                             dst.at[pl.ds(q * 128, 128)], sem)
            for q in range(_BPW // 128)
        ]
        for cp in copies:
            cp.wait()
        pltpu.sync_copy(dst, out_ref.at[pl.ds(base, _BPW)])

    do_table(u_hbm, eu_hbm)
    do_table(i_hbm, ei_hbm)


def _sc_gather(z, u, i):
    mesh = plsc.VectorSubcoreMesh(core_axis_name="c", subcore_axis_name="s")
    f = functools.partial(
        pl.kernel,
        mesh=mesh,
        out_type=(
            jax.ShapeDtypeStruct((BATCH, 2 * EMB), jnp.float32),
            jax.ShapeDtypeStruct((BATCH, 2 * EMB), jnp.float32),
        ),
        scratch_types=[
            pltpu.VMEM((_BPW,), jnp.int32),
            pltpu.VMEM((_BPW, 2 * EMB), jnp.float32),
            pltpu.SemaphoreType.DMA,
        ],
    )(_gather_body)
    return f(z, u, i)


def _mlp_body(eu_ref, ei_ref, w1a_ref, w1b_ref, b1_ref, w2_ref, b2_ref,
              w3_ref, b3_ref, out_ref):
    h = jnp.dot(eu_ref[...], w1a_ref[...], preferred_element_type=jnp.float32)
    h = h + jnp.dot(ei_ref[...], w1b_ref[...],
                    preferred_element_type=jnp.float32)
    h = jnp.maximum(h + b1_ref[...], 0.0)
    h = jnp.maximum(
        jnp.dot(h, w2_ref[...], preferred_element_type=jnp.float32)
        + b2_ref[...], 0.0)
    o = jnp.dot(h, w3_ref[...], preferred_element_type=jnp.float32) + b3_ref[...]
    out_ref[...] = jax.nn.sigmoid(o)


def _tc_mlp(eu, ei, w1, b1, w2, b2, w3, b3):
    blk = 2048
    grid = BATCH // blk
    zpad = jnp.zeros((EMB, 128), jnp.float32)
    w1a = jnp.concatenate([w1[:EMB], zpad], axis=0)
    w1b = jnp.concatenate([zpad, w1[EMB:]], axis=0)
    full = lambda shape: pl.BlockSpec(shape, lambda g: (0, 0))
    out = pl.pallas_call(
        _mlp_body,
        grid=(grid,),
        in_specs=[
            pl.BlockSpec((blk, 2 * EMB), lambda g: (g, 0)),
            pl.BlockSpec((blk, 2 * EMB), lambda g: (g, 0)),
            full((128, 128)),
            full((128, 128)),
            full((1, 128)),
            full((128, 64)),
            full((1, 64)),
            full((64, 1)),
            full((1, 1)),
        ],
        out_specs=pl.BlockSpec((blk, 1), lambda g: (g, 0)),
        out_shape=jax.ShapeDtypeStruct((BATCH, 1), jnp.float32),
        compiler_params=pltpu.CompilerParams(
            dimension_semantics=("arbitrary",)),
    )(eu, ei, w1a, w1b, b1.reshape(1, 128), w2, b2.reshape(1, 64), w3,
      b3.reshape(1, 1))
    return jnp.squeeze(out, axis=-1)


def kernel(u, i, Wu, Wi, W1, b1, W2, b2, W3, b3):
    z = _tc_transpose(Wu.T, Wi.T)
    eu, ei = _sc_gather(z, u.astype(jnp.int32), i.astype(jnp.int32))
    return _tc_mlp(eu, ei, W1, b1, W2, b2, W3, b3)


# speedup vs baseline: 3.0463x; 1.1422x over previous
import functools

import jax
import jax.numpy as jnp
from jax import lax
from jax.experimental import pallas as pl
from jax.experimental.pallas import tpu as pltpu
from jax.experimental.pallas import tpu_sc as plsc

BATCH = 16384
EMB = 64
NROWS = 1000001
_TBLK = 8192
_TGRID = (NROWS + _TBLK - 1) // _TBLK
ZROWS = _TGRID * _TBLK

_info = plsc.get_sparse_core_info()
_NC, _NS = _info.num_cores, _info.num_subcores
_NW = _NC * _NS
_BPW = BATCH // _NW


def _transpose_body(wuT_ref, wiT_ref, eye_ref, z_ref):
    x = jnp.concatenate(
        [wuT_ref[...], wiT_ref[...]], axis=0).astype(jnp.bfloat16)
    dn = (((0,), (0,)), ((), ()))
    z_ref[...] = lax.dot_general(x, eye_ref[...], dn,
                                 preferred_element_type=jnp.float32)


def _tc_transpose(wuT, wiT):
    eye = jnp.eye(2 * EMB, dtype=jnp.bfloat16)
    return pl.pallas_call(
        _transpose_body,
        grid=(_TGRID,),
        in_specs=[
            pl.BlockSpec((EMB, _TBLK), lambda g: (0, g)),
            pl.BlockSpec((EMB, _TBLK), lambda g: (0, g)),
            pl.BlockSpec((2 * EMB, 2 * EMB), lambda g: (0, 0)),
        ],
        out_specs=pl.BlockSpec((_TBLK, 2 * EMB), lambda g: (g, 0)),
        out_shape=jax.ShapeDtypeStruct((ZROWS, 2 * EMB), jnp.float32),
        compiler_params=pltpu.CompilerParams(
            dimension_semantics=("arbitrary",)),
    )(wuT, wiT, eye)


def _gather_body(z_hbm, u_hbm, i_hbm, eu_hbm, ei_hbm, idx_v, dst, sem):
    wid = lax.axis_index("s") * _NC + lax.axis_index("c")
    base = wid * _BPW

    def do_table(idx_hbm, out_ref):
        pltpu.sync_copy(idx_hbm.at[pl.ds(base, _BPW)], idx_v)
        copies = [
            pltpu.async_copy(z_hbm.at[idx_v.at[pl.ds(q * 128, 128)]],
                             dst.at[pl.ds(q * 128, 128)], sem)
            for q in range(_BPW // 128)
        ]
        for cp in copies:
            cp.wait()
        pltpu.sync_copy(dst, out_ref.at[pl.ds(base, _BPW)])

    do_table(u_hbm, eu_hbm)
    do_table(i_hbm, ei_hbm)


def _sc_gather(z, u, i):
    mesh = plsc.VectorSubcoreMesh(core_axis_name="c", subcore_axis_name="s")
    f = functools.partial(
        pl.kernel,
        mesh=mesh,
        out_type=(
            jax.ShapeDtypeStruct((BATCH, 2 * EMB), jnp.float32),
            jax.ShapeDtypeStruct((BATCH, 2 * EMB), jnp.float32),
        ),
        scratch_types=[
            pltpu.VMEM((_BPW,), jnp.int32),
            pltpu.VMEM((_BPW, 2 * EMB), jnp.float32),
            pltpu.SemaphoreType.DMA,
        ],
    )(_gather_body)
    return f(z, u, i)


def _mlp_body(eu_ref, ei_ref, w1a_ref, w1b_ref, b1_ref, w2_ref, b2_ref,
              w3_ref, b3_ref, out_ref):
    h = jnp.dot(eu_ref[...], w1a_ref[...], preferred_element_type=jnp.float32)
    h = h + jnp.dot(ei_ref[...], w1b_ref[...],
                    preferred_element_type=jnp.float32)
    h = jnp.maximum(h + b1_ref[...], 0.0)
    h = jnp.maximum(
        jnp.dot(h, w2_ref[...], preferred_element_type=jnp.float32)
        + b2_ref[...], 0.0)
    o = jnp.dot(h, w3_ref[...], preferred_element_type=jnp.float32) + b3_ref[...]
    out_ref[...] = jax.nn.sigmoid(o)


def _tc_mlp(eu, ei, w1, b1, w2, b2, w3, b3):
    blk = 2048
    grid = BATCH // blk
    zpad = jnp.zeros((EMB, 128), jnp.float32)
    w1a = jnp.concatenate([w1[:EMB], zpad], axis=0)
    w1b = jnp.concatenate([zpad, w1[EMB:]], axis=0)
    full = lambda shape: pl.BlockSpec(shape, lambda g: (0, 0))
    out = pl.pallas_call(
        _mlp_body,
        grid=(grid,),
        in_specs=[
            pl.BlockSpec((blk, 2 * EMB), lambda g: (g, 0)),
            pl.BlockSpec((blk, 2 * EMB), lambda g: (g, 0)),
            full((128, 128)),
            full((128, 128)),
            full((1, 128)),
            full((128, 64)),
            full((1, 64)),
            full((64, 1)),
            full((1, 1)),
        ],
        out_specs=pl.BlockSpec((blk, 1), lambda g: (g, 0)),
        out_shape=jax.ShapeDtypeStruct((BATCH, 1), jnp.float32),
        compiler_params=pltpu.CompilerParams(
            dimension_semantics=("arbitrary",)),
    )(eu, ei, w1a, w1b, b1.reshape(1, 128), w2, b2.reshape(1, 64), w3,
      b3.reshape(1, 1))
    return jnp.squeeze(out, axis=-1)


def kernel(u, i, Wu, Wi, W1, b1, W2, b2, W3, b3):
    z = _tc_transpose(Wu.T, Wi.T)
    eu, ei = _sc_gather(z, u.astype(jnp.int32), i.astype(jnp.int32))
    return _tc_mlp(eu, ei, W1, b1, W2, b2, W3, b3)


# speedup vs baseline: 3.0666x; 1.0067x over previous
import functools

import jax
import jax.numpy as jnp
from jax import lax
from jax.experimental import pallas as pl
from jax.experimental.pallas import tpu as pltpu
from jax.experimental.pallas import tpu_sc as plsc

BATCH = 16384
EMB = 64
NROWS = 1000001
_TBLK = 16384
_TGRID = (NROWS + _TBLK - 1) // _TBLK
ZROWS = _TGRID * _TBLK

_info = plsc.get_sparse_core_info()
_NC, _NS = _info.num_cores, _info.num_subcores
_NW = _NC * _NS
_BPW = BATCH // _NW


def _transpose_body(wuT_ref, wiT_ref, eye_ref, z_ref):
    x = jnp.concatenate(
        [wuT_ref[...], wiT_ref[...]], axis=0).astype(jnp.bfloat16)
    dn = (((0,), (0,)), ((), ()))
    z_ref[...] = lax.dot_general(x, eye_ref[...], dn,
                                 preferred_element_type=jnp.float32)


def _tc_transpose(wuT, wiT):
    eye = jnp.eye(2 * EMB, dtype=jnp.bfloat16)
    return pl.pallas_call(
        _transpose_body,
        grid=(_TGRID,),
        in_specs=[
            pl.BlockSpec((EMB, _TBLK), lambda g: (0, g)),
            pl.BlockSpec((EMB, _TBLK), lambda g: (0, g)),
            pl.BlockSpec((2 * EMB, 2 * EMB), lambda g: (0, 0)),
        ],
        out_specs=pl.BlockSpec((_TBLK, 2 * EMB), lambda g: (g, 0)),
        out_shape=jax.ShapeDtypeStruct((ZROWS, 2 * EMB), jnp.float32),
        compiler_params=pltpu.CompilerParams(
            dimension_semantics=("arbitrary",)),
    )(wuT, wiT, eye)


def _gather_body(z_hbm, u_hbm, i_hbm, eu_hbm, ei_hbm, idx_v, dst, sem):
    wid = lax.axis_index("s") * _NC + lax.axis_index("c")
    base = wid * _BPW

    def do_table(idx_hbm, out_ref):
        pltpu.sync_copy(idx_hbm.at[pl.ds(base, _BPW)], idx_v)
        copies = [
            pltpu.async_copy(z_hbm.at[idx_v.at[pl.ds(q * 128, 128)]],
                             dst.at[pl.ds(q * 128, 128)], sem)
            for q in range(_BPW // 128)
        ]
        for cp in copies:
            cp.wait()
        pltpu.sync_copy(dst, out_ref.at[pl.ds(base, _BPW)])

    do_table(u_hbm, eu_hbm)
    do_table(i_hbm, ei_hbm)


def _sc_gather(z, u, i):
    mesh = plsc.VectorSubcoreMesh(core_axis_name="c", subcore_axis_name="s")
    f = functools.partial(
        pl.kernel,
        mesh=mesh,
        out_type=(
            jax.ShapeDtypeStruct((BATCH, 2 * EMB), jnp.float32),
            jax.ShapeDtypeStruct((BATCH, 2 * EMB), jnp.float32),
        ),
        scratch_types=[
            pltpu.VMEM((_BPW,), jnp.int32),
            pltpu.VMEM((_BPW, 2 * EMB), jnp.float32),
            pltpu.SemaphoreType.DMA,
        ],
    )(_gather_body)
    return f(z, u, i)


def _mlp_body(eu_ref, ei_ref, w1a_ref, w1b_ref, b1_ref, w2_ref, b2_ref,
              w3_ref, b3_ref, out_ref):
    h = jnp.dot(eu_ref[...], w1a_ref[...], preferred_element_type=jnp.float32)
    h = h + jnp.dot(ei_ref[...], w1b_ref[...],
                    preferred_element_type=jnp.float32)
    h = jnp.maximum(h + b1_ref[...], 0.0)
    h = jnp.maximum(
        jnp.dot(h, w2_ref[...], preferred_element_type=jnp.float32)
        + b2_ref[...], 0.0)
    o = jnp.dot(h, w3_ref[...], preferred_element_type=jnp.float32) + b3_ref[...]
    out_ref[...] = jax.nn.sigmoid(o)


def _tc_mlp(eu, ei, w1, b1, w2, b2, w3, b3):
    blk = 2048
    grid = BATCH // blk
    zpad = jnp.zeros((EMB, 128), jnp.float32)
    w1a = jnp.concatenate([w1[:EMB], zpad], axis=0)
    w1b = jnp.concatenate([zpad, w1[EMB:]], axis=0)
    full = lambda shape: pl.BlockSpec(shape, lambda g: (0, 0))
    out = pl.pallas_call(
        _mlp_body,
        grid=(grid,),
        in_specs=[
            pl.BlockSpec((blk, 2 * EMB), lambda g: (g, 0)),
            pl.BlockSpec((blk, 2 * EMB), lambda g: (g, 0)),
            full((128, 128)),
            full((128, 128)),
            full((1, 128)),
            full((128, 64)),
            full((1, 64)),
            full((64, 1)),
            full((1, 1)),
        ],
        out_specs=pl.BlockSpec((blk, 1), lambda g: (g, 0)),
        out_shape=jax.ShapeDtypeStruct((BATCH, 1), jnp.float32),
        compiler_params=pltpu.CompilerParams(
            dimension_semantics=("arbitrary",)),
    )(eu, ei, w1a, w1b, b1.reshape(1, 128), w2, b2.reshape(1, 64), w3,
      b3.reshape(1, 1))
    return jnp.squeeze(out, axis=-1)


def kernel(u, i, Wu, Wi, W1, b1, W2, b2, W3, b3):
    z = _tc_transpose(Wu.T, Wi.T)
    eu, ei = _sc_gather(z, u.astype(jnp.int32), i.astype(jnp.int32))
    return _tc_mlp(eu, ei, W1, b1, W2, b2, W3, b3)
